# baseline (speedup 1.0000x reference)
import functools

import jax
import jax.numpy as jnp
from jax import lax
from jax.experimental import pallas as pl
from jax.experimental.pallas import tpu as pltpu
from jax.experimental.pallas import tpu_sc as plsc

HIDDEN = 768
EXPERTS = 64
TOKENS = 4 * 8192
SEQ_PER_B = 8192
CHUNK_SIZES = (24576, 8192)
TC_BLK = 1024
GRP = 16
NW = 32
CAND = 8


def _tc_body(x_ref, w_ref, b_ref, key_ref, s_ref):
    xb = x_ref[0]
    w = w_ref[...]
    lg = lax.dot_general(w, xb, (((1,), (1,)), ((), ())),
                         preferred_element_type=jnp.float32)
    lg = lg + b_ref[...]
    m = jnp.max(lg, axis=0, keepdims=True)
    s = jnp.sum(jnp.exp(lg - m), axis=0, keepdims=True)
    u = lax.bitcast_convert_type(lg, jnp.int32)
    key = u ^ ((u >> 31) & jnp.int32(0x7FFFFFFF))
    eidx = lax.broadcasted_iota(jnp.int32, (EXPERTS, TC_BLK), 0)
    key = (key & jnp.int32(-64)) | (jnp.int32(63) - eidx)
    g = EXPERTS // 2
    m1 = jnp.maximum(key[:g], key[g:])
    m2 = jnp.minimum(key[:g], key[g:])
    while g > CAND // 2:
        g //= 2
        a1, b1 = m1[:g], m1[g:]
        a2, b2 = m2[:g], m2[g:]
        m2 = jnp.maximum(jnp.minimum(a1, b1), jnp.maximum(a2, b2))
        m1 = jnp.maximum(a1, b1)
    key_ref[...] = jnp.concatenate([m1, m2], axis=0)
    s_ref[...] = s


def _make_tc_project(offset, ct):
    off = offset // TC_BLK
    nb = SEQ_PER_B // TC_BLK
    return pl.pallas_call(
        _tc_body,
        grid=(ct // TC_BLK,),
        in_specs=[
            pl.BlockSpec((1, TC_BLK, HIDDEN),
                         lambda i: ((i + off) // nb, (i + off) % nb, 0)),
            pl.BlockSpec((EXPERTS, HIDDEN), lambda i: (0, 0)),
            pl.BlockSpec((EXPERTS, 1), lambda i: (0, 0)),
        ],
        out_specs=[
            pl.BlockSpec((CAND, TC_BLK), lambda i: (0, i)),
            pl.BlockSpec((1, TC_BLK), lambda i: (0, i)),
        ],
        out_shape=[
            jax.ShapeDtypeStruct((CAND, ct), jnp.int32),
            jax.ShapeDtypeStruct((1, ct), jnp.float32),
        ],
        compiler_params=pltpu.CompilerParams(
            dimension_semantics=("arbitrary",)),
    )


def _unkey(k):
    u = k ^ ((k >> 31) & jnp.int32(0x7FFFFFFF))
    return lax.bitcast_convert_type(u, jnp.float32)


def _make_sc_top2(ct):
    tpw = ct // NW

    @functools.partial(
        pl.kernel,
        mesh=plsc.VectorSubcoreMesh(core_axis_name="c", subcore_axis_name="s"),
        out_type=[
            jax.ShapeDtypeStruct((2, ct), jnp.float32),
            jax.ShapeDtypeStruct((2, ct), jnp.int32),
        ],
        scratch_types=[
            pltpu.VMEM((CAND, tpw), jnp.int32),
            pltpu.VMEM((1, tpw), jnp.float32),
            pltpu.VMEM((2, tpw), jnp.float32),
            pltpu.VMEM((2, tpw), jnp.int32),
        ],
    )
    def _sc_top2(key_hbm, s_hbm, g_hbm, i_hbm, key_v, s_v, g_v, i_v):
        wid = lax.axis_index("s") * 2 + lax.axis_index("c")
        base = wid * tpw
        pltpu.sync_copy(key_hbm.at[:, pl.ds(base, tpw)], key_v)
        pltpu.sync_copy(s_hbm.at[:, pl.ds(base, tpw)], s_v)

        def group(g, carry):
            ts = g * GRP
            m1 = jnp.full((GRP,), jnp.int32(-2147483648))
            m2 = m1
            for e in range(CAND):
                v = key_v[e, pl.ds(ts, GRP)]
                gt1 = v > m1
                gt2 = v > m2
                m2 = jnp.where(gt1, m1, jnp.where(gt2, v, m2))
                m1 = jnp.where(gt1, v, m1)
            i1 = jnp.int32(63) - (m1 & jnp.int32(63))
            i2 = jnp.int32(63) - (m2 & jnp.int32(63))
            v1 = _unkey(m1 & jnp.int32(-64))
            v2 = _unkey(m2 & jnp.int32(-64))
            inv = 1.0 / s_v[0, pl.ds(ts, GRP)]
            g2 = jnp.exp(v2 - v1) * inv
            g_v[0, pl.ds(ts, GRP)] = inv
            g_v[1, pl.ds(ts, GRP)] = g2
            i_v[0, pl.ds(ts, GRP)] = i1
            i_v[1, pl.ds(ts, GRP)] = i2
            return carry

        lax.fori_loop(0, tpw // GRP, group, 0)
        pltpu.sync_copy(g_v, g_hbm.at[:, pl.ds(base, tpw)])
        pltpu.sync_copy(i_v, i_hbm.at[:, pl.ds(base, tpw)])

    return _sc_top2


_offsets = [sum(CHUNK_SIZES[:c]) for c in range(len(CHUNK_SIZES))]
_tc_projects = [_make_tc_project(o, ct) for o, ct in zip(_offsets, CHUNK_SIZES)]
_sc_top2s = [_make_sc_top2(ct) for ct in CHUNK_SIZES]


def kernel(x, W, b):
    b2 = b.reshape(EXPERTS, 1)
    gs, js = [], []
    for c in range(len(CHUNK_SIZES)):
        kt, s = _tc_projects[c](x, W, b2)
        g, i = _sc_top2s[c](kt, s)
        gs.append(g.T)
        js.append(i.T)
    g = jnp.concatenate(gs, axis=0) if len(gs) > 1 else gs[0]
    i = jnp.concatenate(js, axis=0) if len(js) > 1 else js[0]
    bsz, seq = x.shape[0], x.shape[1]
    return g.reshape(bsz, seq, 2), i.reshape(bsz, seq, 2)


# speedup vs baseline: 1.2219x; 1.2219x over previous
import functools

import jax
import jax.numpy as jnp
from jax import lax
from jax.experimental import pallas as pl
from jax.experimental.pallas import tpu as pltpu
from jax.experimental.pallas import tpu_sc as plsc

HIDDEN = 768
EXPERTS = 64
TOKENS = 4 * 8192
SEQ_PER_B = 8192
CHUNK_SIZES = (32768,)
TC_BLK = 2048
GRP = 16
NW = 32
CAND = 8


def _tc_body(x_ref, w_ref, b_ref, key_ref, s_ref):
    xb = x_ref[0]
    w = w_ref[...]
    lg = lax.dot_general(w, xb, (((1,), (1,)), ((), ())),
                         preferred_element_type=jnp.float32)
    lg = lg + b_ref[...]
    m = jnp.max(lg, axis=0, keepdims=True)
    s = jnp.sum(jnp.exp(lg - m), axis=0, keepdims=True)
    u = lax.bitcast_convert_type(lg, jnp.int32)
    key = u ^ ((u >> 31) & jnp.int32(0x7FFFFFFF))
    eidx = lax.broadcasted_iota(jnp.int32, (EXPERTS, TC_BLK), 0)
    key = (key & jnp.int32(-64)) | (jnp.int32(63) - eidx)
    g = EXPERTS // 2
    m1 = jnp.maximum(key[:g], key[g:])
    m2 = jnp.minimum(key[:g], key[g:])
    while g > CAND // 2:
        g //= 2
        a1, b1 = m1[:g], m1[g:]
        a2, b2 = m2[:g], m2[g:]
        m2 = jnp.maximum(jnp.minimum(a1, b1), jnp.maximum(a2, b2))
        m1 = jnp.maximum(a1, b1)
    key_ref[...] = jnp.concatenate([m1, m2], axis=0)
    s_ref[...] = s


def _make_tc_project(offset, ct):
    off = offset // TC_BLK
    nb = SEQ_PER_B // TC_BLK
    return pl.pallas_call(
        _tc_body,
        grid=(ct // TC_BLK,),
        in_specs=[
            pl.BlockSpec((1, TC_BLK, HIDDEN),
                         lambda i: ((i + off) // nb, (i + off) % nb, 0)),
            pl.BlockSpec((EXPERTS, HIDDEN), lambda i: (0, 0)),
            pl.BlockSpec((EXPERTS, 1), lambda i: (0, 0)),
        ],
        out_specs=[
            pl.BlockSpec((CAND, TC_BLK), lambda i: (0, i)),
            pl.BlockSpec((1, TC_BLK), lambda i: (0, i)),
        ],
        out_shape=[
            jax.ShapeDtypeStruct((CAND, ct), jnp.int32),
            jax.ShapeDtypeStruct((1, ct), jnp.float32),
        ],
        compiler_params=pltpu.CompilerParams(
            dimension_semantics=("arbitrary",)),
    )


def _unkey(k):
    u = k ^ ((k >> 31) & jnp.int32(0x7FFFFFFF))
    return lax.bitcast_convert_type(u, jnp.float32)


def _make_sc_top2(ct):
    tpw = ct // NW

    @functools.partial(
        pl.kernel,
        mesh=plsc.VectorSubcoreMesh(core_axis_name="c", subcore_axis_name="s"),
        out_type=[
            jax.ShapeDtypeStruct((2, ct), jnp.float32),
            jax.ShapeDtypeStruct((2, ct), jnp.int32),
        ],
        scratch_types=[
            pltpu.VMEM((CAND, tpw), jnp.int32),
            pltpu.VMEM((1, tpw), jnp.float32),
            pltpu.VMEM((2, tpw), jnp.float32),
            pltpu.VMEM((2, tpw), jnp.int32),
        ],
    )
    def _sc_top2(key_hbm, s_hbm, g_hbm, i_hbm, key_v, s_v, g_v, i_v):
        wid = lax.axis_index("s") * 2 + lax.axis_index("c")
        base = wid * tpw
        pltpu.sync_copy(key_hbm.at[:, pl.ds(base, tpw)], key_v)
        pltpu.sync_copy(s_hbm.at[:, pl.ds(base, tpw)], s_v)

        def group(g, carry):
            ts = g * GRP
            m1 = jnp.full((GRP,), jnp.int32(-2147483648))
            m2 = m1
            for e in range(CAND):
                v = key_v[e, pl.ds(ts, GRP)]
                gt1 = v > m1
                gt2 = v > m2
                m2 = jnp.where(gt1, m1, jnp.where(gt2, v, m2))
                m1 = jnp.where(gt1, v, m1)
            i1 = jnp.int32(63) - (m1 & jnp.int32(63))
            i2 = jnp.int32(63) - (m2 & jnp.int32(63))
            v1 = _unkey(m1 & jnp.int32(-64))
            v2 = _unkey(m2 & jnp.int32(-64))
            inv = 1.0 / s_v[0, pl.ds(ts, GRP)]
            g2 = jnp.exp(v2 - v1) * inv
            g_v[0, pl.ds(ts, GRP)] = inv
            g_v[1, pl.ds(ts, GRP)] = g2
            i_v[0, pl.ds(ts, GRP)] = i1
            i_v[1, pl.ds(ts, GRP)] = i2
            return carry

        lax.fori_loop(0, tpw // GRP, group, 0)
        pltpu.sync_copy(g_v, g_hbm.at[:, pl.ds(base, tpw)])
        pltpu.sync_copy(i_v, i_hbm.at[:, pl.ds(base, tpw)])

    return _sc_top2


_offsets = [sum(CHUNK_SIZES[:c]) for c in range(len(CHUNK_SIZES))]
_tc_projects = [_make_tc_project(o, ct) for o, ct in zip(_offsets, CHUNK_SIZES)]
_sc_top2s = [_make_sc_top2(ct) for ct in CHUNK_SIZES]


def kernel(x, W, b):
    b2 = b.reshape(EXPERTS, 1)
    gs, js = [], []
    for c in range(len(CHUNK_SIZES)):
        kt, s = _tc_projects[c](x, W, b2)
        g, i = _sc_top2s[c](kt, s)
        gs.append(g.T)
        js.append(i.T)
    g = jnp.concatenate(gs, axis=0) if len(gs) > 1 else gs[0]
    i = jnp.concatenate(js, axis=0) if len(js) > 1 else js[0]
    bsz, seq = x.shape[0], x.shape[1]
    return g.reshape(bsz, seq, 2), i.reshape(bsz, seq, 2)


# speedup vs baseline: 1.2823x; 1.0494x over previous
import functools

import jax
import jax.numpy as jnp
from jax import lax
from jax.experimental import pallas as pl
from jax.experimental.pallas import tpu as pltpu
from jax.experimental.pallas import tpu_sc as plsc

HIDDEN = 768
EXPERTS = 64
TOKENS = 4 * 8192
SEQ_PER_B = 8192
CHUNK_SIZES = (32768,)
TC_BLK = 4096
GRP = 16
NW = 32
CAND = 8


def _tc_body(x_ref, w_ref, b_ref, key_ref, s_ref):
    xb = x_ref[0]
    w = w_ref[...]
    lg = lax.dot_general(w, xb, (((1,), (1,)), ((), ())),
                         preferred_element_type=jnp.float32)
    lg = lg + b_ref[...]
    m = jnp.max(lg, axis=0, keepdims=True)
    s = jnp.sum(jnp.exp(lg - m), axis=0, keepdims=True)
    u = lax.bitcast_convert_type(lg, jnp.int32)
    key = u ^ ((u >> 31) & jnp.int32(0x7FFFFFFF))
    eidx = lax.broadcasted_iota(jnp.int32, (EXPERTS, TC_BLK), 0)
    key = (key & jnp.int32(-64)) | (jnp.int32(63) - eidx)
    g = EXPERTS // 2
    m1 = jnp.maximum(key[:g], key[g:])
    m2 = jnp.minimum(key[:g], key[g:])
    while g > CAND // 2:
        g //= 2
        a1, b1 = m1[:g], m1[g:]
        a2, b2 = m2[:g], m2[g:]
        m2 = jnp.maximum(jnp.minimum(a1, b1), jnp.maximum(a2, b2))
        m1 = jnp.maximum(a1, b1)
    key_ref[...] = jnp.concatenate([m1, m2], axis=0)
    s_ref[...] = s


def _make_tc_project(offset, ct):
    off = offset // TC_BLK
    nb = SEQ_PER_B // TC_BLK
    return pl.pallas_call(
        _tc_body,
        grid=(ct // TC_BLK,),
        in_specs=[
            pl.BlockSpec((1, TC_BLK, HIDDEN),
                         lambda i: ((i + off) // nb, (i + off) % nb, 0)),
            pl.BlockSpec((EXPERTS, HIDDEN), lambda i: (0, 0)),
            pl.BlockSpec((EXPERTS, 1), lambda i: (0, 0)),
        ],
        out_specs=[
            pl.BlockSpec((CAND, TC_BLK), lambda i: (0, i)),
            pl.BlockSpec((1, TC_BLK), lambda i: (0, i)),
        ],
        out_shape=[
            jax.ShapeDtypeStruct((CAND, ct), jnp.int32),
            jax.ShapeDtypeStruct((1, ct), jnp.float32),
        ],
        compiler_params=pltpu.CompilerParams(
            dimension_semantics=("arbitrary",)),
    )


def _unkey(k):
    u = k ^ ((k >> 31) & jnp.int32(0x7FFFFFFF))
    return lax.bitcast_convert_type(u, jnp.float32)


def _make_sc_top2(ct):
    tpw = ct // NW

    @functools.partial(
        pl.kernel,
        mesh=plsc.VectorSubcoreMesh(core_axis_name="c", subcore_axis_name="s"),
        out_type=[
            jax.ShapeDtypeStruct((2, ct), jnp.float32),
            jax.ShapeDtypeStruct((2, ct), jnp.int32),
        ],
        scratch_types=[
            pltpu.VMEM((CAND, tpw), jnp.int32),
            pltpu.VMEM((1, tpw), jnp.float32),
            pltpu.VMEM((2, tpw), jnp.float32),
            pltpu.VMEM((2, tpw), jnp.int32),
        ],
    )
    def _sc_top2(key_hbm, s_hbm, g_hbm, i_hbm, key_v, s_v, g_v, i_v):
        wid = lax.axis_index("s") * 2 + lax.axis_index("c")
        base = wid * tpw
        pltpu.sync_copy(key_hbm.at[:, pl.ds(base, tpw)], key_v)
        pltpu.sync_copy(s_hbm.at[:, pl.ds(base, tpw)], s_v)

        def group(g, carry):
            ts = g * GRP
            m1 = jnp.full((GRP,), jnp.int32(-2147483648))
            m2 = m1
            for e in range(CAND):
                v = key_v[e, pl.ds(ts, GRP)]
                gt1 = v > m1
                gt2 = v > m2
                m2 = jnp.where(gt1, m1, jnp.where(gt2, v, m2))
                m1 = jnp.where(gt1, v, m1)
            i1 = jnp.int32(63) - (m1 & jnp.int32(63))
            i2 = jnp.int32(63) - (m2 & jnp.int32(63))
            v1 = _unkey(m1 & jnp.int32(-64))
            v2 = _unkey(m2 & jnp.int32(-64))
            inv = 1.0 / s_v[0, pl.ds(ts, GRP)]
            g2 = jnp.exp(v2 - v1) * inv
            g_v[0, pl.ds(ts, GRP)] = inv
            g_v[1, pl.ds(ts, GRP)] = g2
            i_v[0, pl.ds(ts, GRP)] = i1
            i_v[1, pl.ds(ts, GRP)] = i2
            return carry

        lax.fori_loop(0, tpw // GRP, group, 0)
        pltpu.sync_copy(g_v, g_hbm.at[:, pl.ds(base, tpw)])
        pltpu.sync_copy(i_v, i_hbm.at[:, pl.ds(base, tpw)])

    return _sc_top2


_offsets = [sum(CHUNK_SIZES[:c]) for c in range(len(CHUNK_SIZES))]
_tc_projects = [_make_tc_project(o, ct) for o, ct in zip(_offsets, CHUNK_SIZES)]
_sc_top2s = [_make_sc_top2(ct) for ct in CHUNK_SIZES]


def kernel(x, W, b):
    b2 = b.reshape(EXPERTS, 1)
    gs, js = [], []
    for c in range(len(CHUNK_SIZES)):
        kt, s = _tc_projects[c](x, W, b2)
        g, i = _sc_top2s[c](kt, s)
        gs.append(g.T)
        js.append(i.T)
    g = jnp.concatenate(gs, axis=0) if len(gs) > 1 else gs[0]
    i = jnp.concatenate(js, axis=0) if len(js) > 1 else js[0]
    bsz, seq = x.shape[0], x.shape[1]
    return g.reshape(bsz, seq, 2), i.reshape(bsz, seq, 2)
